# baseline (speedup 1.0000x reference)
import functools
import jax
import jax.numpy as jnp
from jax import lax
from jax.experimental import pallas as pl
from jax.experimental.pallas import tpu as pltpu
from jax.experimental.pallas import tpu_sc as plsc

D_MODEL = 64
SCALE = 8.0
NC, NS, L = 2, 16, 16
NW = NC * NS
B1, B2 = 4096, 200
CH = 128
NBLK = B2
OUT_ROWS = B2 * (D_MODEL // 8)
OUT_COLS = B1 // CH
TILE_W = 8 * CH


@functools.partial(
    pl.kernel,
    out_type=jax.ShapeDtypeStruct((OUT_ROWS, OUT_COLS, TILE_W), jnp.float32),
    mesh=plsc.VectorSubcoreMesh(core_axis_name="c", subcore_axis_name="s"),
    scratch_types=[
        pltpu.VMEM((B2, CH), jnp.int32),
        pltpu.VMEM((CH, D_MODEL), jnp.float32),
        pltpu.VMEM((CH, D_MODEL), jnp.float32),
        pltpu.VMEM((8, TILE_W), jnp.float32),
        pltpu.VMEM((8, TILE_W), jnp.float32),
        pltpu.SemaphoreType.DMA,
        pltpu.SemaphoreType.DMA,
        pltpu.SemaphoreType.DMA,
        pltpu.SemaphoreType.DMA,
    ],
    compiler_params=pltpu.CompilerParams(
        use_tc_tiling_on_sc=False, needs_layout_passes=False
    ),
)
def _embed_sc(xt_hbm, lut_hbm, out_hbm, idx_all, gin0, gin1, buf0, buf1,
              sg0, sg1, sw0, sw1):
    w = lax.axis_index("s") * NC + lax.axis_index("c")
    gin = (gin0, gin1)
    buf = (buf0, buf1)
    sg = (sg0, sg1)
    sw = (sw0, sw1)

    pltpu.sync_copy(xt_hbm.at[:, pl.ds(CH * w, CH)], idx_all)
    pltpu.async_copy(lut_hbm.at[idx_all.at[0]], gin0, sg0)
    pltpu.async_copy(lut_hbm.at[idx_all.at[1]], gin1, sg1)

    lanes = lax.iota(jnp.int32, L)

    def group_body(k, carry):
        for b in range(2):
            b2 = 2 * k + b
            pltpu.make_async_copy(lut_hbm.at[idx_all.at[b2]], gin[b], sg[b]).wait()

            g = gin[b]
            t = buf[b]
            for i in range(8):
                for ds_ in range(8):
                    for v in range(CH // L):
                        bl = lanes + (v * L)
                        col = jnp.full((L,), 8 * i + ds_, jnp.int32)
                        vals = plsc.load_gather(g, [bl, col])
                        t[i, pl.ds(ds_ * CH + v * L, L)] = vals * SCALE

            @pl.when(b2 + 2 < NBLK)
            def _():
                pltpu.async_copy(lut_hbm.at[idx_all.at[b2 + 2]], gin[b], sg[b])

            @pl.when(k >= 1)
            def _():
                pltpu.make_async_copy(
                    t, out_hbm.at[pl.ds(b2 * 8, 8), w], sw[b]
                ).wait()

            pltpu.async_copy(t, out_hbm.at[pl.ds(b2 * 8, 8), w], sw[b])
        return carry

    lax.fori_loop(0, NBLK // 2, group_body, 0)

    for b in range(2):
        b2 = NBLK - 2 + b
        pltpu.make_async_copy(
            buf[b], out_hbm.at[pl.ds(b2 * 8, 8), w], sw[b]
        ).wait()


def kernel(x, lut):
    out3 = _embed_sc(x.T, lut)
    out6 = out3.reshape(B2, 8, OUT_COLS, 8, CH)
    return out6.transpose(2, 4, 0, 1, 3).reshape(B1, B2, D_MODEL)


# speedup vs baseline: 1.2330x; 1.2330x over previous
import functools
import jax
import jax.numpy as jnp
from jax import lax
from jax.experimental import pallas as pl
from jax.experimental.pallas import tpu as pltpu
from jax.experimental.pallas import tpu_sc as plsc

D_MODEL = 64
SCALE = 8.0
NC, NS, L = 2, 16, 16
NW = NC * NS
B1, B2 = 4096, 200
CH = 128
NBLK = B2
OUT_ROWS = B2 * (D_MODEL // 8)
OUT_COLS = B1 // CH
TILE_W = 8 * CH


@functools.partial(
    pl.kernel,
    out_type=jax.ShapeDtypeStruct((OUT_ROWS, OUT_COLS, TILE_W), jnp.float32),
    mesh=plsc.VectorSubcoreMesh(core_axis_name="c", subcore_axis_name="s"),
    scratch_types=[
        pltpu.VMEM((B2, CH), jnp.int32),
        pltpu.VMEM((CH, D_MODEL), jnp.float32),
        pltpu.VMEM((CH, D_MODEL), jnp.float32),
        pltpu.VMEM((8, TILE_W), jnp.float32),
        pltpu.VMEM((8, TILE_W), jnp.float32),
        pltpu.SemaphoreType.DMA,
        pltpu.SemaphoreType.DMA,
        pltpu.SemaphoreType.DMA,
        pltpu.SemaphoreType.DMA,
    ],
    compiler_params=pltpu.CompilerParams(
        use_tc_tiling_on_sc=False, needs_layout_passes=False
    ),
)
def _embed_sc(xt_hbm, lut_hbm, out_hbm, idx_all, gin0, gin1, buf0, buf1,
              sg0, sg1, sw0, sw1):
    w = lax.axis_index("s") * NC + lax.axis_index("c")
    gin = (gin0, gin1)
    buf = (buf0, buf1)
    sg = (sg0, sg1)
    sw = (sw0, sw1)

    pltpu.sync_copy(xt_hbm.at[:, pl.ds(CH * w, CH)], idx_all)
    pltpu.async_copy(lut_hbm.at[idx_all.at[0]], gin0, sg0)
    pltpu.async_copy(lut_hbm.at[idx_all.at[1]], gin1, sg1)

    lanes = lax.iota(jnp.int32, L)
    row_idx = [lanes // 8 + 2 * jj for jj in range(D_MODEL // L)]
    col_base = (lanes % 8) * CH

    def group_body(k, carry):
        for b in range(2):
            b2 = 2 * k + b
            pltpu.make_async_copy(lut_hbm.at[idx_all.at[b2]], gin[b], sg[b]).wait()

            g = gin[b]
            t = buf[b]

            def bl_body(bl, carry2):
                col_idx = col_base + bl
                for jj in range(D_MODEL // L):
                    vals = g[bl, pl.ds(jj * L, L)]
                    plsc.store_scatter(t, [row_idx[jj], col_idx], vals * SCALE)
                return carry2

            lax.fori_loop(0, CH, bl_body, 0)

            @pl.when(b2 + 2 < NBLK)
            def _():
                pltpu.async_copy(lut_hbm.at[idx_all.at[b2 + 2]], gin[b], sg[b])

            @pl.when(k >= 1)
            def _():
                pltpu.make_async_copy(
                    t, out_hbm.at[pl.ds(b2 * 8, 8), w], sw[b]
                ).wait()

            pltpu.async_copy(t, out_hbm.at[pl.ds(b2 * 8, 8), w], sw[b])
        return carry

    lax.fori_loop(0, NBLK // 2, group_body, 0)

    for b in range(2):
        b2 = NBLK - 2 + b
        pltpu.make_async_copy(
            buf[b], out_hbm.at[pl.ds(b2 * 8, 8), w], sw[b]
        ).wait()


def kernel(x, lut):
    out3 = _embed_sc(x.T, lut)
    out6 = out3.reshape(B2, 8, OUT_COLS, 8, CH)
    return out6.transpose(2, 4, 0, 1, 3).reshape(B1, B2, D_MODEL)


# speedup vs baseline: 1.8950x; 1.5369x over previous
import functools
import jax
import jax.numpy as jnp
from jax import lax
from jax.experimental import pallas as pl
from jax.experimental.pallas import tpu as pltpu
from jax.experimental.pallas import tpu_sc as plsc

D_MODEL = 64
SCALE = 8.0
NC, NS, L = 2, 16, 16
NW = NC * NS
B1, B2 = 4096, 200
CH = 128
NBLK = B2
OUT_ROWS = B2 * (D_MODEL // 8)
OUT_COLS = B1 // CH
TILE_W = 8 * CH
PAD_W = CH + 1


@functools.partial(
    pl.kernel,
    out_type=jax.ShapeDtypeStruct((OUT_ROWS, OUT_COLS, 8, CH), jnp.float32),
    mesh=plsc.VectorSubcoreMesh(core_axis_name="c", subcore_axis_name="s"),
    scratch_types=[
        pltpu.VMEM((B2, CH), jnp.int32),
        pltpu.VMEM((CH, D_MODEL), jnp.float32),
        pltpu.VMEM((CH, D_MODEL), jnp.float32),
        pltpu.VMEM((8, 8, PAD_W), jnp.float32),
        pltpu.VMEM((8, 8, PAD_W), jnp.float32),
        pltpu.SemaphoreType.DMA,
        pltpu.SemaphoreType.DMA,
        pltpu.SemaphoreType.DMA,
        pltpu.SemaphoreType.DMA,
    ],
    compiler_params=pltpu.CompilerParams(
        use_tc_tiling_on_sc=False, needs_layout_passes=False
    ),
)
def _embed_sc(xt_hbm, lut_hbm, out_hbm, idx_all, gin0, gin1, buf0, buf1,
              sg0, sg1, sw0, sw1):
    w = lax.axis_index("s") * NC + lax.axis_index("c")
    gin = (gin0, gin1)
    buf = (buf0, buf1)
    sg = (sg0, sg1)
    sw = (sw0, sw1)

    pltpu.sync_copy(xt_hbm.at[:, pl.ds(CH * w, CH)], idx_all)
    pltpu.async_copy(lut_hbm.at[idx_all.at[0]], gin0, sg0)
    pltpu.async_copy(lut_hbm.at[idx_all.at[1]], gin1, sg1)

    lanes = lax.iota(jnp.int32, L)
    row_idx = [lanes // 8 + 2 * jj for jj in range(D_MODEL // L)]
    mid_idx = lanes % 8

    def group_body(k, carry):
        for b in range(2):
            b2 = 2 * k + b
            pltpu.make_async_copy(lut_hbm.at[idx_all.at[b2]], gin[b], sg[b]).wait()

            g = gin[b]
            t = buf[b]

            def bl_body(bl, carry2):
                col_idx = jnp.full((L,), 0, jnp.int32) + bl
                for jj in range(D_MODEL // L):
                    vals = g[bl, pl.ds(jj * L, L)]
                    plsc.store_scatter(
                        t, [row_idx[jj], mid_idx, col_idx], vals * SCALE
                    )
                return carry2

            lax.fori_loop(0, CH, bl_body, 0)

            @pl.when(b2 + 2 < NBLK)
            def _():
                pltpu.async_copy(lut_hbm.at[idx_all.at[b2 + 2]], gin[b], sg[b])

            @pl.when(k >= 1)
            def _():
                pltpu.make_async_copy(
                    t.at[:, :, pl.ds(0, CH)],
                    out_hbm.at[pl.ds(b2 * 8, 8), w], sw[b]
                ).wait()

            pltpu.async_copy(
                t.at[:, :, pl.ds(0, CH)], out_hbm.at[pl.ds(b2 * 8, 8), w], sw[b]
            )
        return carry

    lax.fori_loop(0, NBLK // 2, group_body, 0)

    for b in range(2):
        b2 = NBLK - 2 + b
        pltpu.make_async_copy(
            buf[b].at[:, :, pl.ds(0, CH)],
            out_hbm.at[pl.ds(b2 * 8, 8), w], sw[b]
        ).wait()


def kernel(x, lut):
    out3 = _embed_sc(x.T, lut)
    out6 = out3.reshape(B2, 8, OUT_COLS, 8, CH)
    return out6.transpose(2, 4, 0, 1, 3).reshape(B1, B2, D_MODEL)


# speedup vs baseline: 1.9332x; 1.0202x over previous
import functools
import jax
import jax.numpy as jnp
from jax import lax
from jax.experimental import pallas as pl
from jax.experimental.pallas import tpu as pltpu
from jax.experimental.pallas import tpu_sc as plsc

D_MODEL = 64
SCALE = 8.0
NC, NS, L = 2, 16, 16
NW = NC * NS
B1, B2 = 4096, 200
CH = 128
NBLK = B2
OUT_ROWS = B2 * (D_MODEL // 8)
OUT_COLS = B1 // CH
TILE_W = 8 * CH
PAD_W = CH + 1


@functools.partial(
    pl.kernel,
    out_type=jax.ShapeDtypeStruct((OUT_ROWS, OUT_COLS, 8, CH), jnp.float32),
    mesh=plsc.VectorSubcoreMesh(core_axis_name="c", subcore_axis_name="s"),
    scratch_types=[
        pltpu.VMEM((B2, CH), jnp.int32),
        pltpu.VMEM((CH, D_MODEL), jnp.float32),
        pltpu.VMEM((CH, D_MODEL), jnp.float32),
        pltpu.VMEM((8, 8, PAD_W), jnp.float32),
        pltpu.VMEM((8, 8, PAD_W), jnp.float32),
        pltpu.SemaphoreType.DMA,
        pltpu.SemaphoreType.DMA,
        pltpu.SemaphoreType.DMA,
        pltpu.SemaphoreType.DMA,
    ],
    compiler_params=pltpu.CompilerParams(
        use_tc_tiling_on_sc=False, needs_layout_passes=False
    ),
)
def _embed_sc(xt_hbm, lut_hbm, out_hbm, idx_all, gin0, gin1, buf0, buf1,
              sg0, sg1, sw0, sw1):
    w = lax.axis_index("s") * NC + lax.axis_index("c")
    gin = (gin0, gin1)
    buf = (buf0, buf1)
    sg = (sg0, sg1)
    sw = (sw0, sw1)

    pltpu.sync_copy(xt_hbm.at[:, pl.ds(CH * w, CH)], idx_all)
    pltpu.async_copy(lut_hbm.at[idx_all.at[0]], gin0, sg0)
    pltpu.async_copy(lut_hbm.at[idx_all.at[1]], gin1, sg1)

    lanes = lax.iota(jnp.int32, L)
    row_idx = [lanes // 8 + 2 * jj for jj in range(D_MODEL // L)]
    mid_idx = lanes % 8

    def group_body(k, carry):
        for b in range(2):
            b2 = 2 * k + b
            pltpu.make_async_copy(lut_hbm.at[idx_all.at[b2]], gin[b], sg[b]).wait()

            g = gin[b]
            t = buf[b]

            def bl_body(i4, col0):
                for u in range(4):
                    bl = 4 * i4 + u
                    col_idx = col0 + u
                    for jj in range(D_MODEL // L):
                        vals = g[bl, pl.ds(jj * L, L)]
                        plsc.store_scatter(
                            t, [row_idx[jj], mid_idx, col_idx], vals * SCALE
                        )
                return col0 + 4

            lax.fori_loop(0, CH // 4, bl_body, jnp.zeros((L,), jnp.int32))

            @pl.when(b2 + 2 < NBLK)
            def _():
                pltpu.async_copy(lut_hbm.at[idx_all.at[b2 + 2]], gin[b], sg[b])

            @pl.when(k >= 1)
            def _():
                pltpu.make_async_copy(
                    t.at[:, :, pl.ds(0, CH)],
                    out_hbm.at[pl.ds(b2 * 8, 8), w], sw[b]
                ).wait()

            pltpu.async_copy(
                t.at[:, :, pl.ds(0, CH)], out_hbm.at[pl.ds(b2 * 8, 8), w], sw[b]
            )
        return carry

    lax.fori_loop(0, NBLK // 2, group_body, 0)

    for b in range(2):
        b2 = NBLK - 2 + b
        pltpu.make_async_copy(
            buf[b].at[:, :, pl.ds(0, CH)],
            out_hbm.at[pl.ds(b2 * 8, 8), w], sw[b]
        ).wait()


def kernel(x, lut):
    out3 = _embed_sc(x.T, lut)
    out6 = out3.reshape(B2, 8, OUT_COLS, 8, CH)
    return out6.transpose(2, 4, 0, 1, 3).reshape(B1, B2, D_MODEL)


# speedup vs baseline: 2.8132x; 1.4552x over previous
import functools
import jax
import jax.numpy as jnp
from jax import lax
from jax.experimental import pallas as pl
from jax.experimental.pallas import tpu as pltpu
from jax.experimental.pallas import tpu_sc as plsc

D_MODEL = 64
SCALE = 8.0
NC, NS, L = 2, 16, 16
NW = NC * NS
B1, B2 = 4096, 200
CH = 128
NBLK = B2
OUT_ROWS = B2 * (D_MODEL // 8)
OUT_COLS = B1 // CH
TILE_W = 8 * CH
PAD_W = CH + 1


@functools.partial(
    pl.kernel,
    out_type=jax.ShapeDtypeStruct((OUT_ROWS, OUT_COLS, 8, CH), jnp.float32),
    mesh=plsc.VectorSubcoreMesh(core_axis_name="c", subcore_axis_name="s"),
    scratch_types=[
        pltpu.VMEM((B2, CH), jnp.int32),
        pltpu.VMEM((CH, D_MODEL), jnp.float32),
        pltpu.VMEM((CH, D_MODEL), jnp.float32),
        pltpu.VMEM((8, 8, PAD_W), jnp.float32),
        pltpu.VMEM((8, 8, PAD_W), jnp.float32),
        pltpu.SemaphoreType.DMA,
        pltpu.SemaphoreType.DMA,
        pltpu.SemaphoreType.DMA,
        pltpu.SemaphoreType.DMA,
    ],
    compiler_params=pltpu.CompilerParams(
        use_tc_tiling_on_sc=False, needs_layout_passes=False
    ),
)
def _embed_sc(xt_hbm, lut_hbm, out_hbm, idx_all, gin0, gin1, buf0, buf1,
              sg0, sg1, sw0, sw1):
    w = lax.axis_index("s") * NC + lax.axis_index("c")
    gin = (gin0, gin1)
    buf = (buf0, buf1)
    sg = (sg0, sg1)
    sw = (sw0, sw1)

    pltpu.sync_copy(xt_hbm.at[:, pl.ds(CH * w, CH)], idx_all)
    pltpu.async_copy(lut_hbm.at[idx_all.at[0]], gin0, sg0)
    pltpu.async_copy(lut_hbm.at[idx_all.at[1]], gin1, sg1)

    lanes = lax.iota(jnp.int32, L)
    row_idx = [lanes // 8 + 2 * jj for jj in range(D_MODEL // L)]
    mid_idx = lanes % 8

    def group_body(k, carry):
        for b in range(2):
            b2 = 2 * k + b
            pltpu.make_async_copy(lut_hbm.at[idx_all.at[b2]], gin[b], sg[b]).wait()

            g = gin[b]
            t = buf[b]

            @functools.partial(plsc.parallel_loop, 0, CH, unroll=4)
            def bl_body(bl):
                col_idx = jnp.zeros((L,), jnp.int32) + bl
                for jj in range(D_MODEL // L):
                    vals = g[bl, pl.ds(jj * L, L)]
                    plsc.store_scatter(
                        t, [row_idx[jj], mid_idx, col_idx], vals * SCALE
                    )

            @pl.when(b2 + 2 < NBLK)
            def _():
                pltpu.async_copy(lut_hbm.at[idx_all.at[b2 + 2]], gin[b], sg[b])

            @pl.when(k >= 1)
            def _():
                pltpu.make_async_copy(
                    t.at[:, :, pl.ds(0, CH)],
                    out_hbm.at[pl.ds(b2 * 8, 8), w], sw[b]
                ).wait()

            pltpu.async_copy(
                t.at[:, :, pl.ds(0, CH)], out_hbm.at[pl.ds(b2 * 8, 8), w], sw[b]
            )
        return carry

    lax.fori_loop(0, NBLK // 2, group_body, 0)

    for b in range(2):
        b2 = NBLK - 2 + b
        pltpu.make_async_copy(
            buf[b].at[:, :, pl.ds(0, CH)],
            out_hbm.at[pl.ds(b2 * 8, 8), w], sw[b]
        ).wait()


def kernel(x, lut):
    out3 = _embed_sc(x.T, lut)
    out6 = out3.reshape(B2, 8, OUT_COLS, 8, CH)
    return out6.transpose(2, 4, 0, 1, 3).reshape(B1, B2, D_MODEL)
